# baseline (speedup 1.0000x reference)
import functools

import jax
import jax.numpy as jnp
from jax import lax
from jax.experimental import pallas as pl
from jax.experimental.pallas import tpu as pltpu
from jax.experimental.pallas import tpu_sc as plsc

BATCH = 16384
D = 64
V = 100001
CHUNK = 128
CH = 256
NFULL = V // CH
TAIL = V - NFULL * CH
TAIL16 = TAIL - TAIL % 16
PD = 65
TAILW = 176
NROWS_OUT = NFULL * CH + TAILW

_mesh = plsc.VectorSubcoreMesh(core_axis_name="c", subcore_axis_name="s")
_NC = _mesh.num_cores
_NW = _NC * _mesh.num_subcores
_NROUNDS = NFULL // _NW
_REM = NFULL - _NROUNDS * _NW


def _make_transpose():
  @functools.partial(
      pl.kernel,
      mesh=_mesh,
      compiler_params=pltpu.CompilerParams(needs_layout_passes=False),
      out_type=jax.ShapeDtypeStruct((NROWS_OUT * PD,), jnp.float32),
      scratch_types=[
          pltpu.VMEM((D, CH), jnp.float32),
          pltpu.VMEM((D, CH), jnp.float32),
          pltpu.VMEM((CH * PD,), jnp.float32),
          pltpu.VMEM((CH * PD,), jnp.float32),
          pltpu.SemaphoreType.DMA,
          pltpu.SemaphoreType.DMA,
          pltpu.SemaphoreType.DMA,
          pltpu.SemaphoreType.DMA,
      ],
  )
  def transpose_kernel(
      tt_hbm, tailpad_hbm, out_hbm, slab0, slab1, cb0, cb1, sin0, sin1,
      sout0, sout1
  ):
    wid = lax.axis_index("s") * _NC + lax.axis_index("c")
    lane = lax.iota(jnp.int32, 16)
    slabs = (slab0, slab1)
    cbs = (cb0, cb1)
    sins = (sin0, sin1)
    souts = (sout0, sout1)

    def col0(kk):
      return (kk * _NW + wid) * CH

    def in_copy(kk, par):
      return pltpu.make_async_copy(
          tt_hbm.at[:, pl.ds(col0(kk), CH)], slabs[par], sins[par]
      )

    def tail_in_copy(par):
      return pltpu.make_async_copy(tailpad_hbm, slabs[par], sins[par])

    def out_copy(kk, par):
      return pltpu.make_async_copy(
          cbs[par],
          out_hbm.at[pl.ds(col0(kk) * PD, CH * PD)],
          souts[par],
      )

    def tail_out_copy(par):
      return pltpu.make_async_copy(
          cbs[par].at[pl.ds(0, TAILW * PD)],
          out_hbm.at[pl.ds(NFULL * CH * PD, TAILW * PD)],
          souts[par],
      )

    def do_transpose(slab, cb, ncols16):
      @plsc.parallel_loop(0, ncols16 // 16, unroll=2)
      def _(c16):
        cv = c16 * 16 + lane
        base_idx = cv * PD
        for f in range(D):
          vec = slab[f, pl.ds(c16 * 16, 16)]
          plsc.store_scatter(cb, [base_idx + f], vec)

    def tail_last_cols(slab, cb):
      for c16 in range(TAIL16 // 16, TAILW // 16):
        cv = c16 * 16 + lane
        base_idx = cv * PD
        for f in range(D):
          vec = slab[f, pl.ds(c16 * 16, 16)]
          plsc.store_scatter(cb, [base_idx + f], vec)

    LAST = _NROUNDS
    is_rem = wid < _REM
    is_tail = wid == _REM

    in_copy(0, 0).start()
    npairs = _NROUNDS // 2

    def pair_body(p, _):
      kk0 = p * 2
      kk1 = kk0 + 1
      in_copy(kk1, 1).start()
      in_copy(kk0, 0).wait()
      @pl.when(p >= 1)
      def _():
        out_copy(kk0 - 2, 0).wait()
      do_transpose(slabs[0], cbs[0], CH)
      out_copy(kk0, 0).start()

      @pl.when(p < npairs - 1)
      def _():
        in_copy(kk0 + 2, 0).start()
      @pl.when((p == npairs - 1) & is_rem)
      def _():
        in_copy(LAST, 0).start()
      @pl.when((p == npairs - 1) & is_tail)
      def _():
        tail_in_copy(0).start()
      in_copy(kk1, 1).wait()
      @pl.when(p >= 1)
      def _():
        out_copy(kk1 - 2, 1).wait()
      do_transpose(slabs[1], cbs[1], CH)
      out_copy(kk1, 1).start()
      return ()

    lax.fori_loop(0, npairs, pair_body, ())

    @pl.when(is_rem)
    def _():
      in_copy(LAST, 0).wait()
      out_copy(LAST - 2, 0).wait()
      do_transpose(slabs[0], cbs[0], CH)
      out_copy(LAST, 0).start()

    @pl.when(is_tail)
    def _():
      tail_in_copy(0).wait()
      out_copy(LAST - 2, 0).wait()
      do_transpose(slabs[0], cbs[0], TAIL16)
      tail_last_cols(slabs[0], cbs[0])
      tail_out_copy(0).start()

    out_copy(_NROUNDS - 1, 1).wait()
    @pl.when(is_rem)
    def _():
      out_copy(LAST, 0).wait()
    @pl.when(is_tail)
    def _():
      tail_out_copy(0).wait()

  return transpose_kernel


def _make_gather(b_per_w: int, n_chunks: int):
  @functools.partial(
      pl.kernel,
      mesh=_mesh,
      compiler_params=pltpu.CompilerParams(
          use_tc_tiling_on_sc=False, needs_layout_passes=False
      ),
      out_type=jax.ShapeDtypeStruct((D, BATCH), jnp.float32),
      scratch_types=[
          pltpu.VMEM((b_per_w,), jnp.int32),
          pltpu.VMEM((b_per_w, PD), jnp.float32),
          pltpu.VMEM((D, b_per_w + 1), jnp.float32),
          pltpu.SemaphoreType.DMA,
      ],
  )
  def gather_kernel(idx_hbm, table_hbm, out_hbm, idx_v, rows_v, cols_v, sem):
    wid = lax.axis_index("s") * _NC + lax.axis_index("c")
    base = wid * b_per_w
    lane = lax.iota(jnp.int32, 16)
    pltpu.sync_copy(idx_hbm.at[pl.ds(base, b_per_w)], idx_v)
    gathers = []
    for j in range(n_chunks):
      gathers.append(
          pltpu.async_copy(
              table_hbm.at[idx_v.at[pl.ds(j * CHUNK, CHUNK)]],
              rows_v.at[pl.ds(j * CHUNK, CHUNK)],
              sem,
          )
      )
    for j in range(n_chunks):
      gathers[j].wait()

      @plsc.parallel_loop(j * CHUNK, (j + 1) * CHUNK, unroll=8)
      def _(c):
        cv = jnp.zeros((16,), jnp.int32) + c
        for f0 in range(0, D, 16):
          vals = plsc.load_gather(rows_v, [cv, f0 + lane])
          plsc.store_scatter(cols_v, [f0 + lane, cv], vals)
    pltpu.sync_copy(
        cols_v.at[:, pl.ds(0, b_per_w)], out_hbm.at[:, pl.ds(base, b_per_w)]
    )

  return gather_kernel


def kernel(books, embedding_table):
  b_per_w = BATCH // _NW
  n_chunks = b_per_w // CHUNK
  tt = embedding_table.T
  tailpad = jnp.pad(tt[:, NFULL * CH :], ((0, 0), (0, CH - TAIL)))
  flat = _make_transpose()(tt, tailpad)
  table_lin = flat.reshape(NROWS_OUT, PD)
  out_t = _make_gather(b_per_w, n_chunks)(books, table_lin)
  return out_t.T


# speedup vs baseline: 2.3546x; 2.3546x over previous
import functools

import jax
import jax.numpy as jnp
from jax import lax
from jax.experimental import pallas as pl
from jax.experimental.pallas import tpu as pltpu
from jax.experimental.pallas import tpu_sc as plsc

BATCH = 16384
EMBED_DIM = 64
CHUNK = 128


def _make_gather(b_per_w: int, n_chunks: int):
  mesh = plsc.VectorSubcoreMesh(core_axis_name="c", subcore_axis_name="s")
  nc = mesh.num_cores

  @functools.partial(
      pl.kernel,
      mesh=mesh,
      compiler_params=pltpu.CompilerParams(use_tc_tiling_on_sc=False),
      out_type=jax.ShapeDtypeStruct((BATCH, EMBED_DIM), jnp.float32),
      scratch_types=[
          pltpu.VMEM((b_per_w,), jnp.int32),
          pltpu.VMEM((b_per_w, EMBED_DIM), jnp.float32),
          pltpu.SemaphoreType.DMA,
          pltpu.SemaphoreType.DMA,
      ],
  )
  def gather_kernel(idx_hbm, table_hbm, out_hbm, idx_v, rows_v, gsem, osem):
    wid = lax.axis_index("s") * nc + lax.axis_index("c")
    base = wid * b_per_w
    pltpu.sync_copy(idx_hbm.at[pl.ds(base, b_per_w)], idx_v)
    gathers = []
    for j in range(n_chunks):
      gathers.append(
          pltpu.async_copy(
              table_hbm.at[idx_v.at[pl.ds(j * CHUNK, CHUNK)]],
              rows_v.at[pl.ds(j * CHUNK, CHUNK)],
              gsem,
          )
      )
    writes = []
    for j in range(n_chunks):
      gathers[j].wait()
      writes.append(
          pltpu.async_copy(
              rows_v.at[pl.ds(j * CHUNK, CHUNK)],
              out_hbm.at[pl.ds(base + j * CHUNK, CHUNK)],
              osem,
          )
      )
    for w in writes:
      w.wait()

  return gather_kernel


def kernel(books, embedding_table):
  info = plsc.get_sparse_core_info()
  num_workers = info.num_cores * info.num_subcores
  b_per_w = BATCH // num_workers
  n_chunks = b_per_w // CHUNK
  return _make_gather(b_per_w, n_chunks)(books, embedding_table)
